# Initial kernel scaffold; baseline (speedup 1.0000x reference)
import jax
import jax.numpy as jnp
from jax.experimental import pallas as pl


def kernel(input, emb_table, pos_table):
    raise NotImplementedError("write your pallas kernel here")


# speedup vs baseline: 1.5780x; 1.5780x over previous
import functools

import jax
import jax.numpy as jnp
from jax import lax
from jax.experimental import pallas as pl
from jax.experimental.pallas import tpu as pltpu
from jax.experimental.pallas import tpu_sc as plsc

B = 4096
S = 200
EMB = 64
HALF = S // 2
NC = 2
NS = 16
NW = NC * NS
ROWS_PER_W = B // NW


def _body(inp_hbm, emb_hbm, pos_hbm, out_hbm, idx2, buf, posv, sem):
    cid = lax.axis_index("c")
    sid = lax.axis_index("s")
    wid = sid * NC + cid

    pltpu.sync_copy(pos_hbm, posv)

    def row_body(r, carry):
        b = wid * ROWS_PER_W + r
        pltpu.sync_copy(inp_hbm.at[b], idx2)
        cp0 = pltpu.async_copy(emb_hbm.at[idx2.at[0]], buf.at[pl.ds(0, HALF)], sem)
        cp1 = pltpu.async_copy(emb_hbm.at[idx2.at[1]], buf.at[pl.ds(HALF, HALF)], sem)
        cp0.wait()
        cp1.wait()

        chunks = [(o, 0, 16) for o in range(0, 96, 16)] + [(84, 12, 4)]
        for h in range(2):
            for off, lane0, cnt in chunks:
                toks = idx2[h, pl.ds(off, 16)]
                for li in range(lane0, lane0 + cnt):
                    sg = h * HALF + off + li
                    f = jnp.where(toks[li] == 0, 0.0, 1.0).astype(jnp.float32)
                    for j in range(EMB // 16):
                        pv = posv[sg, pl.ds(16 * j, 16)]
                        plsc.addupdate(buf.at[sg, pl.ds(16 * j, 16)], pv * f)

        pltpu.sync_copy(buf, out_hbm.at[b])
        return carry

    lax.fori_loop(0, ROWS_PER_W, row_body, 0)


@functools.partial(
    pl.kernel,
    out_type=jax.ShapeDtypeStruct((B, S, EMB), jnp.float32),
    mesh=plsc.VectorSubcoreMesh(core_axis_name="c", subcore_axis_name="s"),
    compiler_params=pltpu.CompilerParams(use_tc_tiling_on_sc=False),
    scratch_types=[
        pltpu.VMEM((2, HALF), jnp.int32),
        pltpu.VMEM((S, EMB), jnp.float32),
        pltpu.VMEM((S, EMB), jnp.float32),
        pltpu.SemaphoreType.DMA,
    ],
)
def _sc_embed(inp_hbm, emb_hbm, pos_hbm, out_hbm, idx2, buf, posv, sem):
    _body(inp_hbm, emb_hbm, pos_hbm, out_hbm, idx2, buf, posv, sem)


def kernel(input, emb_table, pos_table):
    inp3 = input.astype(jnp.int32).reshape(B, 2, HALF)
    pos_shift = pos_table[1:S + 1]
    return _sc_embed(inp3, emb_table, pos_shift)


# speedup vs baseline: 1.6943x; 1.0737x over previous
import functools

import jax
import jax.numpy as jnp
from jax import lax
from jax.experimental import pallas as pl
from jax.experimental.pallas import tpu as pltpu
from jax.experimental.pallas import tpu_sc as plsc

B = 4096
S = 200
EMB = 64
NC = 2
NS = 16
NW = NC * NS
BC = B // 128
UNITS_PER_W = S * BC // NW
NBUF = 4


def _unit_body(t, emb_hbm, inp_hbm, pose_hbm, out_hbm, idx_all,
               rbuf, tbuf, prow, irow, gsem, wbsem, wid, descs):
    u = wid * UNITS_PER_W + t
    s = u // BC
    bc = u % BC

    pltpu.make_async_copy(emb_hbm.at[idx_all.at[t]], rbuf, gsem).wait()
    pltpu.make_async_copy(pose_hbm.at[s], prow, gsem).wait()
    pltpu.make_async_copy(inp_hbm.at[wid, t], irow, gsem).wait()

    def gbody(g, c):
        toks = irow[pl.ds(16 * g, 16)]
        fvec = jnp.where(toks == 0, jnp.float32(0), jnp.float32(1))
        bvec = lax.iota(jnp.int32, 16) + 16 * g
        for d in range(EMB):
            pv = prow[pl.ds(16 * d, 16)]
            dvec = jnp.full((16,), d, jnp.int32)
            gv = plsc.load_gather(rbuf, [bvec, dvec])
            off = (d // 8) * 1024 + (d % 8) * 128 + 16 * g
            tbuf[pl.ds(off, 16)] = gv + pv * fvec
        return c

    lax.fori_loop(0, 8, gbody, 0)

    for dt in range(8):
        descs.append(pltpu.async_copy(
            tbuf.at[pl.ds(dt * 1024, 1024)], out_hbm.at[s, dt, bc], wbsem))

    t_n = t + NBUF

    @pl.when(t_n < UNITS_PER_W)
    def _():
        u_n = wid * UNITS_PER_W + t_n
        s_n = u_n // BC
        pltpu.async_copy(emb_hbm.at[idx_all.at[t_n]], rbuf, gsem)
        pltpu.async_copy(pose_hbm.at[s_n], prow, gsem)
        pltpu.async_copy(inp_hbm.at[wid, t_n], irow, gsem)


@functools.partial(
    pl.kernel,
    out_type=jax.ShapeDtypeStruct((S, EMB // 8, BC, 1024), jnp.float32),
    mesh=plsc.VectorSubcoreMesh(core_axis_name="c", subcore_axis_name="s"),
    compiler_params=pltpu.CompilerParams(
        use_tc_tiling_on_sc=False, needs_layout_passes=False),
    scratch_types=[
        pltpu.VMEM((UNITS_PER_W, 128), jnp.int32),
        pltpu.VMEM((128, EMB), jnp.float32),
        pltpu.VMEM((128, EMB), jnp.float32),
        pltpu.VMEM((128, EMB), jnp.float32),
        pltpu.VMEM((128, EMB), jnp.float32),
        pltpu.VMEM((8 * 1024,), jnp.float32),
        pltpu.VMEM((8 * 1024,), jnp.float32),
        pltpu.VMEM((8 * 1024,), jnp.float32),
        pltpu.VMEM((8 * 1024,), jnp.float32),
        pltpu.VMEM((16 * EMB,), jnp.float32),
        pltpu.VMEM((16 * EMB,), jnp.float32),
        pltpu.VMEM((16 * EMB,), jnp.float32),
        pltpu.VMEM((16 * EMB,), jnp.float32),
        pltpu.VMEM((128,), jnp.int32),
        pltpu.VMEM((128,), jnp.int32),
        pltpu.VMEM((128,), jnp.int32),
        pltpu.VMEM((128,), jnp.int32),
        pltpu.SemaphoreType.DMA,
        pltpu.SemaphoreType.DMA,
        pltpu.SemaphoreType.DMA,
        pltpu.SemaphoreType.DMA,
        pltpu.SemaphoreType.DMA,
    ],
)
def _sc_embed(inp_hbm, emb_hbm, pose_hbm, out_hbm, idx_all,
              rb0, rb1, rb2, rb3, tb0, tb1, tb2, tb3,
              pr0, pr1, pr2, pr3, ir0, ir1, ir2, ir3,
              g0, g1, g2, g3, wbsem):
    cid = lax.axis_index("c")
    sid = lax.axis_index("s")
    wid = sid * NC + cid
    rbufs = (rb0, rb1, rb2, rb3)
    tbufs = (tb0, tb1, tb2, tb3)
    prows = (pr0, pr1, pr2, pr3)
    irows = (ir0, ir1, ir2, ir3)
    gsems = (g0, g1, g2, g3)

    pltpu.sync_copy(inp_hbm.at[wid], idx_all)

    for k in range(NBUF):
        u0 = wid * UNITS_PER_W + k
        pltpu.async_copy(emb_hbm.at[idx_all.at[k]], rbufs[k], gsems[k])
        pltpu.async_copy(pose_hbm.at[u0 // BC], prows[k], gsems[k])
        pltpu.async_copy(inp_hbm.at[wid, k], irows[k], gsems[k])

    def round_body(i, carry):
        descs = []
        for k in range(NBUF):
            _unit_body(i * NBUF + k, emb_hbm, inp_hbm, pose_hbm, out_hbm,
                       idx_all, rbufs[k], tbufs[k], prows[k], irows[k],
                       gsems[k], wbsem, wid, descs)
        for dsc in descs:
            dsc.wait()
        return carry

    lax.fori_loop(0, UNITS_PER_W // NBUF, round_body, 0)


def kernel(input, emb_table, pos_table):
    inp_r = input.astype(jnp.int32).T.reshape(NW, UNITS_PER_W, 128)
    pos_exp = jnp.broadcast_to(
        pos_table[1:S + 1][:, :, None], (S, EMB, 16)).reshape(S, EMB * 16)
    out4 = _sc_embed(inp_r, emb_table, pos_exp)
    v = out4.reshape(S, EMB // 8, BC, 8, 128)
    return v.transpose(2, 4, 0, 1, 3).reshape(B, S, EMB)


# speedup vs baseline: 2.3203x; 1.3694x over previous
import functools

import jax
import jax.numpy as jnp
from jax import lax
from jax.experimental import pallas as pl
from jax.experimental.pallas import tpu as pltpu
from jax.experimental.pallas import tpu_sc as plsc

B = 4096
S = 200
EMB = 64
NC = 2
NS = 16
NW = NC * NS
BC = B // 128
UNITS_PER_W = S * BC // NW
NBUF = 4


def _unit_body(t, emb_hbm, inp_hbm, pose_hbm, out_hbm, idx_all,
               rbuf, tbuf, prow, irow, gsem, wbsem, wid, descs):
    u = wid * UNITS_PER_W + t
    s = u // BC
    bc = u % BC

    pltpu.make_async_copy(emb_hbm.at[idx_all.at[t]], rbuf, gsem).wait()
    pltpu.make_async_copy(pose_hbm.at[s], prow, gsem).wait()
    pltpu.make_async_copy(inp_hbm.at[wid, t], irow, gsem).wait()

    @plsc.parallel_loop(0, 8, step=1, unroll=2)
    def _gloop(g):
        toks = irow[pl.ds(16 * g, 16)]
        fvec = jnp.where(toks == 0, jnp.float32(0), jnp.float32(1))
        bvec = lax.iota(jnp.int32, 16) + 16 * g
        for d in range(EMB):
            pv = prow[pl.ds(16 * d, 16)]
            dvec = jnp.full((16,), d, jnp.int32)
            gv = plsc.load_gather(rbuf, [bvec, dvec])
            off = (d // 8) * 1024 + (d % 8) * 128 + 16 * g
            tbuf[pl.ds(off, 16)] = gv + pv * fvec

    for dt in range(8):
        descs.append(pltpu.async_copy(
            tbuf.at[pl.ds(dt * 1024, 1024)], out_hbm.at[s, dt, bc], wbsem))

    t_n = t + NBUF

    @pl.when(t_n < UNITS_PER_W)
    def _():
        u_n = wid * UNITS_PER_W + t_n
        s_n = u_n // BC
        pltpu.async_copy(emb_hbm.at[idx_all.at[t_n]], rbuf, gsem)
        pltpu.async_copy(pose_hbm.at[s_n], prow, gsem)
        pltpu.async_copy(inp_hbm.at[wid, t_n], irow, gsem)


@functools.partial(
    pl.kernel,
    out_type=jax.ShapeDtypeStruct((S, EMB // 8, BC, 1024), jnp.float32),
    mesh=plsc.VectorSubcoreMesh(core_axis_name="c", subcore_axis_name="s"),
    compiler_params=pltpu.CompilerParams(
        use_tc_tiling_on_sc=False, needs_layout_passes=False),
    scratch_types=[
        pltpu.VMEM((UNITS_PER_W, 128), jnp.int32),
        pltpu.VMEM((128, EMB), jnp.float32),
        pltpu.VMEM((128, EMB), jnp.float32),
        pltpu.VMEM((128, EMB), jnp.float32),
        pltpu.VMEM((128, EMB), jnp.float32),
        pltpu.VMEM((8 * 1024,), jnp.float32),
        pltpu.VMEM((8 * 1024,), jnp.float32),
        pltpu.VMEM((8 * 1024,), jnp.float32),
        pltpu.VMEM((8 * 1024,), jnp.float32),
        pltpu.VMEM((16 * EMB,), jnp.float32),
        pltpu.VMEM((16 * EMB,), jnp.float32),
        pltpu.VMEM((16 * EMB,), jnp.float32),
        pltpu.VMEM((16 * EMB,), jnp.float32),
        pltpu.VMEM((128,), jnp.int32),
        pltpu.VMEM((128,), jnp.int32),
        pltpu.VMEM((128,), jnp.int32),
        pltpu.VMEM((128,), jnp.int32),
        pltpu.SemaphoreType.DMA,
        pltpu.SemaphoreType.DMA,
        pltpu.SemaphoreType.DMA,
        pltpu.SemaphoreType.DMA,
        pltpu.SemaphoreType.DMA,
    ],
)
def _sc_embed(inp_hbm, emb_hbm, pose_hbm, out_hbm, idx_all,
              rb0, rb1, rb2, rb3, tb0, tb1, tb2, tb3,
              pr0, pr1, pr2, pr3, ir0, ir1, ir2, ir3,
              g0, g1, g2, g3, wbsem):
    cid = lax.axis_index("c")
    sid = lax.axis_index("s")
    wid = sid * NC + cid
    rbufs = (rb0, rb1, rb2, rb3)
    tbufs = (tb0, tb1, tb2, tb3)
    prows = (pr0, pr1, pr2, pr3)
    irows = (ir0, ir1, ir2, ir3)
    gsems = (g0, g1, g2, g3)

    pltpu.sync_copy(inp_hbm.at[wid], idx_all)

    for k in range(NBUF):
        u0 = wid * UNITS_PER_W + k
        pltpu.async_copy(emb_hbm.at[idx_all.at[k]], rbufs[k], gsems[k])
        pltpu.async_copy(pose_hbm.at[u0 // BC], prows[k], gsems[k])
        pltpu.async_copy(inp_hbm.at[wid, k], irows[k], gsems[k])

    def round_body(i, carry):
        descs = []
        for k in range(NBUF):
            _unit_body(i * NBUF + k, emb_hbm, inp_hbm, pose_hbm, out_hbm,
                       idx_all, rbufs[k], tbufs[k], prows[k], irows[k],
                       gsems[k], wbsem, wid, descs)
        for dsc in descs:
            dsc.wait()
        return carry

    lax.fori_loop(0, UNITS_PER_W // NBUF, round_body, 0)


def kernel(input, emb_table, pos_table):
    inp_r = input.astype(jnp.int32).T.reshape(NW, UNITS_PER_W, 128)
    pos_exp = jnp.broadcast_to(
        pos_table[1:S + 1][:, :, None], (S, EMB, 16)).reshape(S, EMB * 16)
    out4 = _sc_embed(inp_r, emb_table, pos_exp)
    v = out4.reshape(S, EMB // 8, BC, 8, 128)
    return v.transpose(2, 4, 0, 1, 3).reshape(B, S, EMB)


# speedup vs baseline: 2.6719x; 1.1515x over previous
import functools

import jax
import jax.numpy as jnp
from jax import lax
from jax.experimental import pallas as pl
from jax.experimental.pallas import tpu as pltpu
from jax.experimental.pallas import tpu_sc as plsc

B = 4096
S = 200
EMB = 64
NC = 2
NS = 16
NW = NC * NS
ST = S // 8
NBUF = 4


def _unit_body(s, inp_hbm, emb_hbm, pose_hbm, out_hbm, idx_all,
               rbuf, tbuf, prow, irow, gsem, wbsem, wid, descs):
    pltpu.make_async_copy(emb_hbm.at[idx_all.at[s]], rbuf, gsem).wait()
    pltpu.make_async_copy(pose_hbm.at[s], prow, gsem).wait()
    pltpu.make_async_copy(inp_hbm.at[s // 8, wid, s % 8], irow, gsem).wait()

    fvecs = []
    bvecs = []
    for g in range(8):
        toks = irow[pl.ds(16 * g, 16)]
        fvecs.append(jnp.where(toks == 0, jnp.float32(0), jnp.float32(1)))
        bvecs.append(lax.iota(jnp.int32, 16) + 16 * g)

    @plsc.parallel_loop(0, EMB, step=1, unroll=4)
    def _dloop(d):
        pv = prow[pl.ds(16 * d, 16)]
        dvec = jnp.full((16,), d, jnp.int32)
        base = (d // 8) * 1024 + (d % 8) * 128
        for g in range(8):
            gv = plsc.load_gather(rbuf, [bvecs[g], dvec])
            tbuf[pl.ds(base + 16 * g, 16)] = gv + pv * fvecs[g]

    for dt in range(8):
        descs.append(pltpu.async_copy(
            tbuf.at[pl.ds(dt * 1024, 1024)], out_hbm.at[s, dt, wid], wbsem))

    s_n = s + NBUF

    @pl.when(s_n < S)
    def _():
        pltpu.async_copy(emb_hbm.at[idx_all.at[s_n]], rbuf, gsem)
        pltpu.async_copy(pose_hbm.at[s_n], prow, gsem)
        pltpu.async_copy(inp_hbm.at[s_n // 8, wid, s_n % 8], irow, gsem)


@functools.partial(
    pl.kernel,
    out_type=jax.ShapeDtypeStruct((S, EMB // 8, NW, 1024), jnp.float32),
    mesh=plsc.VectorSubcoreMesh(core_axis_name="c", subcore_axis_name="s"),
    compiler_params=pltpu.CompilerParams(
        use_tc_tiling_on_sc=False, needs_layout_passes=False),
    scratch_types=[
        pltpu.VMEM((S, 128), jnp.int32),
        pltpu.VMEM((128, EMB), jnp.float32),
        pltpu.VMEM((128, EMB), jnp.float32),
        pltpu.VMEM((128, EMB), jnp.float32),
        pltpu.VMEM((128, EMB), jnp.float32),
        pltpu.VMEM((8 * 1024,), jnp.float32),
        pltpu.VMEM((8 * 1024,), jnp.float32),
        pltpu.VMEM((8 * 1024,), jnp.float32),
        pltpu.VMEM((8 * 1024,), jnp.float32),
        pltpu.VMEM((16 * EMB,), jnp.float32),
        pltpu.VMEM((16 * EMB,), jnp.float32),
        pltpu.VMEM((16 * EMB,), jnp.float32),
        pltpu.VMEM((16 * EMB,), jnp.float32),
        pltpu.VMEM((128,), jnp.int32),
        pltpu.VMEM((128,), jnp.int32),
        pltpu.VMEM((128,), jnp.int32),
        pltpu.VMEM((128,), jnp.int32),
        pltpu.SemaphoreType.DMA,
        pltpu.SemaphoreType.DMA,
        pltpu.SemaphoreType.DMA,
        pltpu.SemaphoreType.DMA,
        pltpu.SemaphoreType.DMA,
    ],
)
def _sc_embed(inp_hbm, emb_hbm, pose_hbm, out_hbm, idx_all,
              rb0, rb1, rb2, rb3, tb0, tb1, tb2, tb3,
              pr0, pr1, pr2, pr3, ir0, ir1, ir2, ir3,
              g0, g1, g2, g3, wbsem):
    cid = lax.axis_index("c")
    sid = lax.axis_index("s")
    wid = sid * NC + cid
    rbufs = (rb0, rb1, rb2, rb3)
    tbufs = (tb0, tb1, tb2, tb3)
    prows = (pr0, pr1, pr2, pr3)
    irows = (ir0, ir1, ir2, ir3)
    gsems = (g0, g1, g2, g3)

    for ts in range(ST):
        pltpu.sync_copy(inp_hbm.at[ts, wid], idx_all.at[pl.ds(ts * 8, 8)])

    for k in range(NBUF):
        pltpu.async_copy(emb_hbm.at[idx_all.at[k]], rbufs[k], gsems[k])
        pltpu.async_copy(pose_hbm.at[k], prows[k], gsems[k])
        pltpu.async_copy(inp_hbm.at[k // 8, wid, k % 8], irows[k], gsems[k])

    def round_body(i, carry):
        descs = []
        for k in range(NBUF):
            _unit_body(i * NBUF + k, inp_hbm, emb_hbm, pose_hbm, out_hbm,
                       idx_all, rbufs[k], tbufs[k], prows[k], irows[k],
                       gsems[k], wbsem, wid, descs)
        for dsc in descs:
            dsc.wait()
        return carry

    lax.fori_loop(0, S // NBUF, round_body, 0)


def kernel(input, emb_table, pos_table):
    inp4 = (input.astype(jnp.int32).T
            .reshape(ST, 8, NW, 128).transpose(0, 2, 1, 3))
    pos_exp = jnp.broadcast_to(
        pos_table[1:S + 1][:, :, None], (S, EMB, 16)).reshape(S, EMB * 16)
    out4 = _sc_embed(inp4, emb_table, pos_exp)
    v = out4.reshape(S, EMB // 8, NW, 8, 128)
    return v.transpose(2, 4, 0, 1, 3).reshape(B, S, EMB)


# speedup vs baseline: 2.8046x; 1.0497x over previous
import functools

import jax
import jax.numpy as jnp
from jax import lax
from jax.experimental import pallas as pl
from jax.experimental.pallas import tpu as pltpu
from jax.experimental.pallas import tpu_sc as plsc

B = 4096
S = 200
EMB = 64
NC = 2
NS = 16
NW = NC * NS
ST = S // 8
NBUF = 4


def _unit_body(s, inp_hbm, emb_hbm, pose_hbm, out_hbm, idx_all,
               rbuf, tbuf, prow, irow, gsem, wbsem, wid, descs):
    pltpu.make_async_copy(emb_hbm.at[idx_all.at[s]], rbuf, gsem).wait()
    pltpu.make_async_copy(pose_hbm.at[s], prow, gsem).wait()
    pltpu.make_async_copy(inp_hbm.at[s // 8, wid, s % 8], irow, gsem).wait()

    fvecs = []
    bvecs = []
    for g in range(8):
        toks = irow[pl.ds(16 * g, 16)]
        fvecs.append(jnp.where(toks == 0, jnp.float32(0), jnp.float32(1)))
        bvecs.append(lax.iota(jnp.int32, 16) + 16 * g)

    @plsc.parallel_loop(0, EMB, step=1, unroll=4)
    def _dloop(d):
        pv = prow[pl.ds(16 * d, 16)]
        dvec = jnp.full((16,), d, jnp.int32)
        base = (d // 8) * 1024 + (d % 8) * 128
        for g in range(8):
            gv = plsc.load_gather(rbuf, [bvecs[g], dvec])
            tbuf[pl.ds(base + 16 * g, 16)] = gv + pv * fvecs[g]

    for dt in range(8):
        descs.append(pltpu.async_copy(
            tbuf.at[pl.ds(dt * 1024, 1024)], out_hbm.at[s, dt, wid], wbsem))

    s_n = s + NBUF

    @pl.when(s_n < S)
    def _():
        pltpu.async_copy(emb_hbm.at[idx_all.at[s_n]], rbuf, gsem)
        pltpu.async_copy(pose_hbm.at[s_n], prow, gsem)
        pltpu.async_copy(inp_hbm.at[s_n // 8, wid, s_n % 8], irow, gsem)


@functools.partial(
    pl.kernel,
    out_type=jax.ShapeDtypeStruct((S, EMB // 8, NW, 1024), jnp.float32),
    mesh=plsc.VectorSubcoreMesh(core_axis_name="c", subcore_axis_name="s"),
    compiler_params=pltpu.CompilerParams(
        use_tc_tiling_on_sc=False, needs_layout_passes=False),
    scratch_types=[
        pltpu.VMEM((S, 128), jnp.int32),
        pltpu.VMEM((128, 128), jnp.float32),
        pltpu.VMEM((128, 128), jnp.float32),
        pltpu.VMEM((128, 128), jnp.float32),
        pltpu.VMEM((128, 128), jnp.float32),
        pltpu.VMEM((8 * 1024,), jnp.float32),
        pltpu.VMEM((8 * 1024,), jnp.float32),
        pltpu.VMEM((8 * 1024,), jnp.float32),
        pltpu.VMEM((8 * 1024,), jnp.float32),
        pltpu.VMEM((16 * EMB,), jnp.float32),
        pltpu.VMEM((16 * EMB,), jnp.float32),
        pltpu.VMEM((16 * EMB,), jnp.float32),
        pltpu.VMEM((16 * EMB,), jnp.float32),
        pltpu.VMEM((128,), jnp.int32),
        pltpu.VMEM((128,), jnp.int32),
        pltpu.VMEM((128,), jnp.int32),
        pltpu.VMEM((128,), jnp.int32),
        pltpu.SemaphoreType.DMA,
        pltpu.SemaphoreType.DMA,
        pltpu.SemaphoreType.DMA,
        pltpu.SemaphoreType.DMA,
        pltpu.SemaphoreType.DMA,
    ],
)
def _sc_embed(inp_hbm, emb_hbm, pose_hbm, out_hbm, idx_all,
              rb0, rb1, rb2, rb3, tb0, tb1, tb2, tb3,
              pr0, pr1, pr2, pr3, ir0, ir1, ir2, ir3,
              g0, g1, g2, g3, wbsem):
    cid = lax.axis_index("c")
    sid = lax.axis_index("s")
    wid = sid * NC + cid
    rbufs = (rb0, rb1, rb2, rb3)
    tbufs = (tb0, tb1, tb2, tb3)
    prows = (pr0, pr1, pr2, pr3)
    irows = (ir0, ir1, ir2, ir3)
    gsems = (g0, g1, g2, g3)

    for ts in range(ST):
        pltpu.sync_copy(inp_hbm.at[ts, wid], idx_all.at[pl.ds(ts * 8, 8)])

    for k in range(NBUF):
        pltpu.async_copy(emb_hbm.at[idx_all.at[k]], rbufs[k], gsems[k])
        pltpu.async_copy(pose_hbm.at[k], prows[k], gsems[k])
        pltpu.async_copy(inp_hbm.at[k // 8, wid, k % 8], irows[k], gsems[k])

    def round_body(i, carry):
        descs = []
        for k in range(NBUF):
            _unit_body(i * NBUF + k, inp_hbm, emb_hbm, pose_hbm, out_hbm,
                       idx_all, rbufs[k], tbufs[k], prows[k], irows[k],
                       gsems[k], wbsem, wid, descs)
        for dsc in descs:
            dsc.wait()
        return carry

    lax.fori_loop(0, S // NBUF, round_body, 0)


def kernel(input, emb_table, pos_table):
    inp4 = (input.astype(jnp.int32).T
            .reshape(ST, 8, NW, 128).transpose(0, 2, 1, 3))
    emb_p = jnp.pad(emb_table, ((0, 0), (0, EMB)))
    pos_exp = jnp.broadcast_to(
        pos_table[1:S + 1][:, :, None], (S, EMB, 16)).reshape(S, EMB * 16)
    out4 = _sc_embed(inp4, emb_p, pos_exp)
    v = out4.reshape(S, EMB // 8, NW, 8, 128)
    return v.transpose(2, 4, 0, 1, 3).reshape(B, S, EMB)


# speedup vs baseline: 2.8459x; 1.0147x over previous
import functools

import jax
import jax.numpy as jnp
from jax import lax
from jax.experimental import pallas as pl
from jax.experimental.pallas import tpu as pltpu
from jax.experimental.pallas import tpu_sc as plsc

B = 4096
S = 200
EMB = 64
NC = 2
NS = 16
NW = NC * NS
ST = S // 8
NBUF = 4


def _drain_wb(s_p, tbuf, out_hbm, wbsem, wid):
    for dt in range(8):
        pltpu.make_async_copy(
            tbuf.at[pl.ds(dt * 1024, 1024)],
            out_hbm.at[s_p, dt, wid], wbsem).wait()


def _unit_body(s, inp_hbm, inps_hbm, emb_hbm, pose_hbm, out_hbm, idx_all,
               rbuf, tbuf, prow, irow, gsem, wbsem, wid, wait_wb):
    pltpu.make_async_copy(emb_hbm.at[idx_all.at[s]], rbuf, gsem).wait()
    pltpu.make_async_copy(pose_hbm.at[s], prow, gsem).wait()
    pltpu.make_async_copy(inp_hbm.at[s // 8, wid, s % 8], irow, gsem).wait()

    if wait_wb:
        _drain_wb(s - NBUF, tbuf, out_hbm, wbsem, wid)

    fvecs = []
    bvecs = []
    for g in range(8):
        toks = irow[pl.ds(16 * g, 16)]
        fvecs.append(jnp.where(toks == 0, jnp.float32(0), jnp.float32(1)))
        bvecs.append(lax.iota(jnp.int32, 16) + 16 * g)

    @plsc.parallel_loop(0, EMB, step=1, unroll=4)
    def _dloop(d):
        pv = prow[pl.ds(16 * d, 16)]
        dvec = jnp.full((16,), d, jnp.int32)
        base = (d // 8) * 1024 + (d % 8) * 128
        for g in range(8):
            gv = plsc.load_gather(rbuf, [bvecs[g], dvec])
            tbuf[pl.ds(base + 16 * g, 16)] = gv + pv * fvecs[g]

    for dt in range(8):
        pltpu.async_copy(
            tbuf.at[pl.ds(dt * 1024, 1024)], out_hbm.at[s, dt, wid], wbsem)

    s_n = s + NBUF

    @pl.when(s_n < S)
    def _():
        pltpu.async_copy(emb_hbm.at[idx_all.at[s_n]], rbuf, gsem)
        pltpu.async_copy(pose_hbm.at[s_n], prow, gsem)
        pltpu.async_copy(inp_hbm.at[s_n // 8, wid, s_n % 8], irow, gsem)


@functools.partial(
    pl.kernel,
    out_type=jax.ShapeDtypeStruct((S, EMB // 8, NW, 1024), jnp.float32),
    mesh=plsc.VectorSubcoreMesh(core_axis_name="c", subcore_axis_name="s"),
    compiler_params=pltpu.CompilerParams(
        use_tc_tiling_on_sc=False, needs_layout_passes=False),
    scratch_types=[
        pltpu.VMEM((S, 128), jnp.int32),
        pltpu.VMEM((128, EMB), jnp.float32),
        pltpu.VMEM((128, EMB), jnp.float32),
        pltpu.VMEM((128, EMB), jnp.float32),
        pltpu.VMEM((128, EMB), jnp.float32),
        pltpu.VMEM((8 * 1024,), jnp.float32),
        pltpu.VMEM((8 * 1024,), jnp.float32),
        pltpu.VMEM((8 * 1024,), jnp.float32),
        pltpu.VMEM((8 * 1024,), jnp.float32),
        pltpu.VMEM((16 * EMB,), jnp.float32),
        pltpu.VMEM((16 * EMB,), jnp.float32),
        pltpu.VMEM((16 * EMB,), jnp.float32),
        pltpu.VMEM((16 * EMB,), jnp.float32),
        pltpu.VMEM((128,), jnp.int32),
        pltpu.VMEM((128,), jnp.int32),
        pltpu.VMEM((128,), jnp.int32),
        pltpu.VMEM((128,), jnp.int32),
        pltpu.SemaphoreType.DMA,
        pltpu.SemaphoreType.DMA,
        pltpu.SemaphoreType.DMA,
        pltpu.SemaphoreType.DMA,
        pltpu.SemaphoreType.DMA,
        pltpu.SemaphoreType.DMA,
        pltpu.SemaphoreType.DMA,
        pltpu.SemaphoreType.DMA,
    ],
)
def _sc_embed(inp_hbm, inps_hbm, emb_hbm, pose_hbm, out_hbm, idx_all,
              rb0, rb1, rb2, rb3, tb0, tb1, tb2, tb3,
              pr0, pr1, pr2, pr3, ir0, ir1, ir2, ir3,
              g0, g1, g2, g3, w0, w1, w2, w3):
    cid = lax.axis_index("c")
    sid = lax.axis_index("s")
    wid = sid * NC + cid
    rbufs = (rb0, rb1, rb2, rb3)
    tbufs = (tb0, tb1, tb2, tb3)
    prows = (pr0, pr1, pr2, pr3)
    irows = (ir0, ir1, ir2, ir3)
    gsems = (g0, g1, g2, g3)
    wsems = (w0, w1, w2, w3)

    for ts in range(ST):
        pltpu.sync_copy(inps_hbm.at[ts, wid], idx_all.at[pl.ds(ts * 8, 8)])

    for k in range(NBUF):
        pltpu.async_copy(emb_hbm.at[idx_all.at[k]], rbufs[k], gsems[k])
        pltpu.async_copy(pose_hbm.at[k], prows[k], gsems[k])
        pltpu.async_copy(inp_hbm.at[k // 8, wid, k % 8], irows[k], gsems[k])

    def make_round(wait_wb):
        def round_body(i, carry):
            for k in range(NBUF):
                _unit_body(i * NBUF + k, inp_hbm, inps_hbm, emb_hbm, pose_hbm,
                           out_hbm, idx_all, rbufs[k], tbufs[k], prows[k],
                           irows[k], gsems[k], wsems[k], wid, wait_wb)
            return carry
        return round_body

    make_round(False)(0, 0)
    lax.fori_loop(1, S // NBUF, make_round(True), 0)

    for k in range(NBUF):
        _drain_wb(S - NBUF + k, tbufs[k], out_hbm, wsems[k], wid)


def kernel(input, emb_table, pos_table):
    inp_i = input.astype(jnp.int32)
    inp4 = inp_i.T.reshape(ST, 8, NW, 128).transpose(0, 2, 1, 3)
    inps4 = (inp_i << 1).T.reshape(ST, 8, NW, 128).transpose(0, 2, 1, 3)
    emb2 = jnp.pad(emb_table, ((0, 0), (0, EMB))).reshape(2 * 1000000, EMB)
    pos_exp = jnp.broadcast_to(
        pos_table[1:S + 1][:, :, None], (S, EMB, 16)).reshape(S, EMB * 16)
    out4 = _sc_embed(inp4, inps4, emb2, pos_exp)
    v = out4.reshape(S, EMB // 8, NW, 8, 128)
    return v.transpose(2, 4, 0, 1, 3).reshape(B, S, EMB)


# speedup vs baseline: 4.8105x; 1.6903x over previous
import functools

import jax
import jax.numpy as jnp
from jax import lax
from jax.experimental import pallas as pl
from jax.experimental.pallas import tpu as pltpu
from jax.experimental.pallas import tpu_sc as plsc

B = 4096
S = 200
EMB = 64
NC = 2
NS = 16
NW = NC * NS
ST = S // 8
NBUF = 4


def _drain_wb(s_p, tbuf, out_hbm, wbsem, wid):
    for dt in range(8):
        pltpu.make_async_copy(
            tbuf.at[pl.ds(dt * 1024, 1024)],
            out_hbm.at[s_p, dt, wid], wbsem).wait()


def _unit_body(s, inp_hbm, inps_hbm, emb_hbm, pose_hbm, out_hbm, idx_all,
               rbuf, rpad, tbuf, prow, irow, gsem, wbsem, wid, wait_wb):
    pltpu.make_async_copy(emb_hbm.at[idx_all.at[s]], rbuf, gsem).wait()
    pltpu.make_async_copy(pose_hbm.at[s], prow, gsem).wait()
    pltpu.make_async_copy(inp_hbm.at[s // 8, wid, s % 8], irow, gsem).wait()

    if wait_wb:
        _drain_wb(s - NBUF, tbuf, out_hbm, wbsem, wid)

    @plsc.parallel_loop(0, 128, step=1, unroll=4)
    def _bloop(b):
        for q in range(4):
            rpad[pl.ds(b * 65 + 16 * q, 16)] = rbuf[b, pl.ds(16 * q, 16)]

    fvecs = []
    avecs = []
    for g in range(8):
        toks = irow[pl.ds(16 * g, 16)]
        fvecs.append(jnp.where(toks == 0, jnp.float32(0), jnp.float32(1)))
        avecs.append((lax.iota(jnp.int32, 16) + 16 * g) * 65)

    @plsc.parallel_loop(0, EMB, step=1, unroll=4)
    def _dloop(d):
        pv = prow[pl.ds(16 * d, 16)]
        base = (d // 8) * 1024 + (d % 8) * 128
        for g in range(8):
            gv = plsc.load_gather(rpad, [avecs[g] + d])
            tbuf[pl.ds(base + 16 * g, 16)] = gv + pv * fvecs[g]

    for dt in range(8):
        pltpu.async_copy(
            tbuf.at[pl.ds(dt * 1024, 1024)], out_hbm.at[s, dt, wid], wbsem)

    s_n = s + NBUF

    @pl.when(s_n < S)
    def _():
        pltpu.async_copy(emb_hbm.at[idx_all.at[s_n]], rbuf, gsem)
        pltpu.async_copy(pose_hbm.at[s_n], prow, gsem)
        pltpu.async_copy(inp_hbm.at[s_n // 8, wid, s_n % 8], irow, gsem)


@functools.partial(
    pl.kernel,
    out_type=jax.ShapeDtypeStruct((S, EMB // 8, NW, 1024), jnp.float32),
    mesh=plsc.VectorSubcoreMesh(core_axis_name="c", subcore_axis_name="s"),
    compiler_params=pltpu.CompilerParams(
        use_tc_tiling_on_sc=False, needs_layout_passes=False),
    scratch_types=[
        pltpu.VMEM((S, 128), jnp.int32),
        pltpu.VMEM((128, EMB), jnp.float32),
        pltpu.VMEM((128, EMB), jnp.float32),
        pltpu.VMEM((128, EMB), jnp.float32),
        pltpu.VMEM((128, EMB), jnp.float32),
        pltpu.VMEM((128 * 65,), jnp.float32),
        pltpu.VMEM((128 * 65,), jnp.float32),
        pltpu.VMEM((128 * 65,), jnp.float32),
        pltpu.VMEM((128 * 65,), jnp.float32),
        pltpu.VMEM((8 * 1024,), jnp.float32),
        pltpu.VMEM((8 * 1024,), jnp.float32),
        pltpu.VMEM((8 * 1024,), jnp.float32),
        pltpu.VMEM((8 * 1024,), jnp.float32),
        pltpu.VMEM((16 * EMB,), jnp.float32),
        pltpu.VMEM((16 * EMB,), jnp.float32),
        pltpu.VMEM((16 * EMB,), jnp.float32),
        pltpu.VMEM((16 * EMB,), jnp.float32),
        pltpu.VMEM((128,), jnp.int32),
        pltpu.VMEM((128,), jnp.int32),
        pltpu.VMEM((128,), jnp.int32),
        pltpu.VMEM((128,), jnp.int32),
        pltpu.SemaphoreType.DMA,
        pltpu.SemaphoreType.DMA,
        pltpu.SemaphoreType.DMA,
        pltpu.SemaphoreType.DMA,
        pltpu.SemaphoreType.DMA,
        pltpu.SemaphoreType.DMA,
        pltpu.SemaphoreType.DMA,
        pltpu.SemaphoreType.DMA,
    ],
)
def _sc_embed(inp_hbm, inps_hbm, emb_hbm, pose_hbm, out_hbm, idx_all,
              rb0, rb1, rb2, rb3, rp0, rp1, rp2, rp3, tb0, tb1, tb2, tb3,
              pr0, pr1, pr2, pr3, ir0, ir1, ir2, ir3,
              g0, g1, g2, g3, w0, w1, w2, w3):
    cid = lax.axis_index("c")
    sid = lax.axis_index("s")
    wid = sid * NC + cid
    rbufs = (rb0, rb1, rb2, rb3)
    rpads = (rp0, rp1, rp2, rp3)
    tbufs = (tb0, tb1, tb2, tb3)
    prows = (pr0, pr1, pr2, pr3)
    irows = (ir0, ir1, ir2, ir3)
    gsems = (g0, g1, g2, g3)
    wsems = (w0, w1, w2, w3)

    for ts in range(ST):
        pltpu.sync_copy(inps_hbm.at[ts, wid], idx_all.at[pl.ds(ts * 8, 8)])

    for k in range(NBUF):
        pltpu.async_copy(emb_hbm.at[idx_all.at[k]], rbufs[k], gsems[k])
        pltpu.async_copy(pose_hbm.at[k], prows[k], gsems[k])
        pltpu.async_copy(inp_hbm.at[k // 8, wid, k % 8], irows[k], gsems[k])

    def make_round(wait_wb):
        def round_body(i, carry):
            for k in range(NBUF):
                _unit_body(i * NBUF + k, inp_hbm, inps_hbm, emb_hbm, pose_hbm,
                           out_hbm, idx_all, rbufs[k], rpads[k], tbufs[k],
                           prows[k], irows[k], gsems[k], wsems[k], wid,
                           wait_wb)
            return carry
        return round_body

    make_round(False)(0, 0)
    lax.fori_loop(1, S // NBUF, make_round(True), 0)

    for k in range(NBUF):
        _drain_wb(S - NBUF + k, tbufs[k], out_hbm, wsems[k], wid)


def kernel(input, emb_table, pos_table):
    inp_i = input.astype(jnp.int32)
    inp4 = inp_i.T.reshape(ST, 8, NW, 128).transpose(0, 2, 1, 3)
    inps4 = (inp_i << 1).T.reshape(ST, 8, NW, 128).transpose(0, 2, 1, 3)
    emb2 = jnp.pad(emb_table, ((0, 0), (0, EMB))).reshape(2 * 1000000, EMB)
    pos_exp = jnp.broadcast_to(
        pos_table[1:S + 1][:, :, None], (S, EMB, 16)).reshape(S, EMB * 16)
    out4 = _sc_embed(inp4, inps4, emb2, pos_exp)
    v = out4.reshape(S, EMB // 8, NW, 8, 128)
    return v.transpose(2, 4, 0, 1, 3).reshape(B, S, EMB)
